# Initial kernel scaffold; baseline (speedup 1.0000x reference)
import jax
import jax.numpy as jnp
from jax.experimental import pallas as pl


def kernel(x, edge_index, batch, Wg1, bg1, Wg2, bg2, Wf1, bf1, Wf2, bf2):
    raise NotImplementedError("write your pallas kernel here")


# speedup vs baseline: 27.7345x; 27.7345x over previous
import functools

import jax
import jax.numpy as jnp
from jax import lax
from jax.experimental import pallas as pl
from jax.experimental.pallas import tpu as pltpu
from jax.experimental.pallas import tpu_sc as plsc

N = 100000
E = 1600000
B = 1024
NPAD = 100352
RPT = NPAD // 16
EW = 2000
EW2 = 1000
F32 = jnp.float32

_MESH = plsc.VectorSubcoreMesh(core_axis_name="c", subcore_axis_name="s")


def _fill(ref, n, value):
    vec = jnp.full((16,), value, F32)

    def body(i, carry):
        ref[pl.ds(i * 16, 16)] = vec
        return carry

    lax.fori_loop(0, n // 16, body, 0)


@functools.partial(
    pl.kernel,
    out_type=jax.ShapeDtypeStruct((2, NPAD), F32),
    mesh=_MESH,
    compiler_params=pltpu.CompilerParams(use_tc_tiling_on_sc=False),
    scratch_types=[
        pltpu.VMEM((EW,), jnp.int32),
        pltpu.VMEM((EW,), F32),
        pltpu.VMEM((RPT,), F32),
        pltpu.VMEM_SHARED((NPAD,), F32),
    ],
)
def _deg_kernel(dst_hbm, deg_hbm, dst_v, ones_v, zero_v, acc):
    c = lax.axis_index("c")
    s = lax.axis_index("s")
    _fill(zero_v, RPT, 0.0)
    _fill(ones_v, EW, 1.0)
    sl = pl.ds(s * RPT, RPT)
    pltpu.sync_copy(zero_v, acc.at[sl])
    plsc.subcore_barrier()

    share = E // 32
    base = (c * 16 + s) * share

    def win(i, carry):
        pltpu.sync_copy(dst_hbm.at[pl.ds(base + i * EW, EW)], dst_v)
        pltpu.sync_copy(ones_v, acc.at[dst_v], add=True)
        return carry

    lax.fori_loop(0, share // EW, win, 0)
    plsc.subcore_barrier()
    pltpu.sync_copy(acc.at[sl], deg_hbm.at[c, sl])


@functools.partial(
    pl.kernel,
    out_type=jax.ShapeDtypeStruct((2, 2 * NPAD), F32),
    mesh=_MESH,
    compiler_params=pltpu.CompilerParams(use_tc_tiling_on_sc=False),
    scratch_types=[
        pltpu.VMEM((EW,), jnp.int32),
        pltpu.VMEM((EW,), jnp.int32),
        pltpu.VMEM((EW,), jnp.int32),
        pltpu.VMEM((EW,), F32),
        pltpu.VMEM((2 * RPT,), F32),
        pltpu.VMEM_SHARED((2 * NPAD,), F32),
    ],
)
def _l1_kernel(src_hbm, dst_hbm, xzf_hbm, t_hbm,
               src_v, dst_v, idx_v, val_v, zero_v, acc):
    c = lax.axis_index("c")
    s = lax.axis_index("s")
    _fill(zero_v, 2 * RPT, 0.0)
    sl = pl.ds(s * 2 * RPT, 2 * RPT)
    pltpu.sync_copy(zero_v, acc.at[sl])
    plsc.subcore_barrier()

    share = E // 32
    base = (c * 16 + s) * share

    def dbl(col_ref, off):
        def g(i, carry):
            v16 = col_ref[pl.ds(i * 16, 16)]
            idx_v[pl.ds(i * 16, 16)] = v16 * 2 + off
            return carry
        lax.fori_loop(0, EW // 16, g, 0)

    def win(i, carry):
        off = base + i * EW
        pltpu.sync_copy(src_hbm.at[pl.ds(off, EW)], src_v)
        pltpu.sync_copy(dst_hbm.at[pl.ds(off, EW)], dst_v)
        for col in (0, 1):
            dbl(src_v, col)
            pltpu.sync_copy(xzf_hbm.at[idx_v], val_v)
            dbl(dst_v, col)
            pltpu.sync_copy(val_v, acc.at[idx_v], add=True)
        return carry

    lax.fori_loop(0, share // EW, win, 0)
    plsc.subcore_barrier()
    pltpu.sync_copy(acc.at[sl], t_hbm.at[c, sl])


@functools.partial(
    pl.kernel,
    out_type=jax.ShapeDtypeStruct((2, NPAD, 16), F32),
    mesh=_MESH,
    compiler_params=pltpu.CompilerParams(use_tc_tiling_on_sc=False),
    scratch_types=[
        pltpu.VMEM((EW2,), jnp.int32),
        pltpu.VMEM((EW2,), jnp.int32),
        pltpu.VMEM((EW2, 16), F32),
        pltpu.VMEM_SHARED((NPAD, 16), F32),
    ],
)
def _l2_kernel(srcs_hbm, dst_hbm, z_hbm, zz_hbm, t_hbm,
               src_v, dst_v, rows_v, acc):
    c = lax.axis_index("c")
    s = lax.axis_index("s")
    sl = pl.ds(s * RPT, RPT)
    pltpu.sync_copy(zz_hbm, acc.at[sl])
    plsc.subcore_barrier()

    share = E // 16
    base = s * share

    def win(i, carry):
        off = base + i * EW2
        pltpu.sync_copy(srcs_hbm.at[c, pl.ds(off, EW2)], src_v)
        pltpu.sync_copy(dst_hbm.at[pl.ds(off, EW2)], dst_v)
        pltpu.sync_copy(z_hbm.at[src_v], rows_v)
        pltpu.sync_copy(rows_v, acc.at[dst_v], add=True)
        return carry

    lax.fori_loop(0, share // EW2, win, 0)
    plsc.subcore_barrier()
    pltpu.sync_copy(acc.at[sl], t_hbm.at[c, sl])


SEG = NPAD // 32
SUB = SEG // 4
ACC_N = 2050 * 16


@functools.partial(
    pl.kernel,
    out_type=jax.ShapeDtypeStruct((32, 32768), F32),
    mesh=_MESH,
    compiler_params=pltpu.CompilerParams(use_tc_tiling_on_sc=False),
    scratch_types=[
        pltpu.VMEM((SUB,), jnp.int32),
        pltpu.VMEM((SUB,), F32),
        pltpu.VMEM((SUB * 16,), F32),
        pltpu.VMEM((SUB * 16,), F32),
        pltpu.VMEM((SUB * 16,), F32),
        pltpu.VMEM((SUB * 16,), F32),
        pltpu.VMEM((ACC_N,), F32),
        pltpu.VMEM((32,), F32),
    ],
)
def _pool_kernel(batch_hbm, dinv_hbm, t_hbm, z_hbm, bg2_hbm, p_hbm,
                 bat_v, dinv_v, ta_v, tb_v, za_v, zb_v, acc, bg2_v):
    c = lax.axis_index("c")
    s = lax.axis_index("s")
    w = c * 16 + s
    _fill(acc, ACC_N, -jnp.inf)
    pltpu.sync_copy(bg2_hbm, bg2_v)
    bga = bg2_v[pl.ds(0, 16)]
    bgb = bg2_v[pl.ds(16, 16)]
    nbase = w * SEG

    def sub(k, carry):
        off = nbase + k * SUB
        osl = pl.ds(off, SUB)
        fsl = pl.ds(off * 16, SUB * 16)
        fsl_b = pl.ds((NPAD + off) * 16, SUB * 16)
        pltpu.sync_copy(batch_hbm.at[osl], bat_v)
        pltpu.sync_copy(dinv_hbm.at[osl], dinv_v)
        pltpu.sync_copy(t_hbm.at[fsl], ta_v)
        pltpu.sync_copy(t_hbm.at[fsl_b], tb_v)
        pltpu.sync_copy(z_hbm.at[fsl], za_v)
        pltpu.sync_copy(z_hbm.at[fsl_b], zb_v)

        def grp(g, carry2):
            b16 = bat_v[pl.ds(g * 16, 16)]
            dv16 = dinv_v[pl.ds(g * 16, 16)]
            for j in range(16):
                rsl = pl.ds((g * 16 + j) * 16, 16)
                b = b16[j]
                dv = dv16[j]
                boff = b * 32
                oa = jnp.maximum(dv * (ta_v[rsl] + za_v[rsl]) + bga, 0.0)
                acc[pl.ds(boff, 16)] = jnp.maximum(acc[pl.ds(boff, 16)], oa)
                ob = jnp.maximum(dv * (tb_v[rsl] + zb_v[rsl]) + bgb, 0.0)
                acc[pl.ds(boff + 16, 16)] = jnp.maximum(
                    acc[pl.ds(boff + 16, 16)], ob)
            return carry2

        lax.fori_loop(0, SUB // 16, grp, 0)
        return carry

    lax.fori_loop(0, SEG // SUB, sub, 0)
    pltpu.sync_copy(acc.at[pl.ds(0, 32768)], p_hbm.at[w])


def _dot(a, b):
    return lax.dot_general(a, b, (((1,), (0,)), ((), ())),
                           preferred_element_type=F32,
                           precision=lax.Precision.HIGHEST)


def _prep_body(d_ref, xz_ref, dinv_ref, xzs_ref):
    d = d_ref[...]
    deg = d[0] + d[1] + 1.0
    dv = 1.0 / jnp.sqrt(deg)
    dinv_ref[...] = dv
    xzs_ref[...] = xz_ref[...] * dv


_prep = pl.pallas_call(
    _prep_body,
    grid=(98,),
    in_specs=[pl.BlockSpec((2, 1024, 1), lambda i: (0, i, 0)),
              pl.BlockSpec((1024, 2), lambda i: (i, 0))],
    out_specs=[pl.BlockSpec((1024, 1), lambda i: (i, 0)),
               pl.BlockSpec((1024, 2), lambda i: (i, 0))],
    out_shape=[jax.ShapeDtypeStruct((NPAD, 1), F32),
               jax.ShapeDtypeStruct((NPAD, 2), F32)],
)


def _mid_body(t_ref, xz_ref, dinv_ref, w1_ref, b1_ref, w2_ref, z_ref):
    dv = dinv_ref[...]
    t = t_ref[...]
    u = dv * (t[0] + t[1] + xz_ref[...])
    h1 = jnp.maximum(_dot(u, w1_ref[...]) + b1_ref[...], 0.0)
    z2 = _dot(h1, w2_ref[...]) * dv
    z_ref[0] = z2[:, :16]
    z_ref[1] = z2[:, 16:]


_mid = pl.pallas_call(
    _mid_body,
    grid=(98,),
    in_specs=[pl.BlockSpec((2, 1024, 2), lambda i: (0, i, 0)),
              pl.BlockSpec((1024, 2), lambda i: (i, 0)),
              pl.BlockSpec((1024, 1), lambda i: (i, 0)),
              pl.BlockSpec((2, 64), lambda i: (0, 0)),
              pl.BlockSpec((1, 64), lambda i: (0, 0)),
              pl.BlockSpec((64, 32), lambda i: (0, 0))],
    out_specs=pl.BlockSpec((2, 1024, 16), lambda i: (0, i, 0)),
    out_shape=jax.ShapeDtypeStruct((2, NPAD, 16), F32),
)


def _fc_body(p_ref, wf1_ref, bf1_ref, wf2_ref, bf2_ref, o_ref, pool_ref):
    i = pl.program_id(0)

    @pl.when(i == 0)
    def _():
        pool_ref[...] = jnp.full((1024, 32), -jnp.inf, F32)

    pool_ref[...] = jnp.maximum(pool_ref[...], p_ref[...][0])

    @pl.when(i == 31)
    def _():
        o = jnp.maximum(_dot(pool_ref[...], wf1_ref[...]) + bf1_ref[...], 0.0)
        o_ref[...] = jnp.maximum(_dot(o, wf2_ref[...]) + bf2_ref[...], 0.0)


_fc = pl.pallas_call(
    _fc_body,
    grid=(32,),
    in_specs=[pl.BlockSpec((1, 1024, 32), lambda i: (i, 0, 0)),
              pl.BlockSpec((32, 32), lambda i: (0, 0)),
              pl.BlockSpec((1, 32), lambda i: (0, 0)),
              pl.BlockSpec((32, 28), lambda i: (0, 0)),
              pl.BlockSpec((1, 28), lambda i: (0, 0))],
    out_specs=pl.BlockSpec((1024, 28), lambda i: (0, 0)),
    out_shape=jax.ShapeDtypeStruct((1024, 28), F32),
    scratch_shapes=[pltpu.VMEM((1024, 32), F32)],
)


def kernel(x, edge_index, batch, Wg1, bg1, Wg2, bg2, Wf1, bf1, Wf2, bf2):
    src_e = edge_index[0]
    dst_e = edge_index[1]
    deg = _deg_kernel(dst_e)
    xp = jnp.concatenate([x, jnp.zeros((NPAD - N, 2), F32)], axis=0)
    dinv, xz = _prep(deg.reshape(2, NPAD, 1), xp)
    t1f = _l1_kernel(src_e, dst_e, xz.reshape(NPAD * 2))
    t1 = t1f.reshape(2, NPAD, 2)
    z = _mid(t1, xz, dinv, Wg1, bg1.reshape(1, 64), Wg2)
    srcs = jnp.stack([src_e, src_e + NPAD])
    t2 = _l2_kernel(srcs, dst_e, z.reshape(2 * NPAD, 16),
                    jnp.zeros((RPT, 16), F32))
    batp = jnp.concatenate([batch, jnp.full((NPAD - N,), B, jnp.int32)])
    p = _pool_kernel(batp, dinv.reshape(NPAD), t2.reshape(2 * NPAD * 16),
                     z.reshape(2 * NPAD * 16), bg2)
    return _fc(p.reshape(32, 1024, 32), Wf1, bf1.reshape(1, 32),
               Wf2, bf2.reshape(1, 28))


# speedup vs baseline: 38.6865x; 1.3949x over previous
import functools

import jax
import jax.numpy as jnp
from jax import lax
from jax.experimental import pallas as pl
from jax.experimental.pallas import tpu as pltpu
from jax.experimental.pallas import tpu_sc as plsc

N = 100000
E = 1600000
B = 1024
NPAD = 100352
RPT = NPAD // 16
EW = 2000
EW2 = 800
F32 = jnp.float32

_MESH = plsc.VectorSubcoreMesh(core_axis_name="c", subcore_axis_name="s")


def _fill(ref, n, value):
    vec = jnp.full((16,), value, F32)

    def body(i, carry):
        ref[pl.ds(i * 16, 16)] = vec
        return carry

    lax.fori_loop(0, n // 16, body, 0)


@functools.partial(
    pl.kernel,
    out_type=jax.ShapeDtypeStruct((2, NPAD), F32),
    mesh=_MESH,
    compiler_params=pltpu.CompilerParams(use_tc_tiling_on_sc=False),
    scratch_types=[
        pltpu.VMEM((EW,), jnp.int32),
        pltpu.VMEM((EW,), F32),
        pltpu.VMEM((RPT,), F32),
        pltpu.VMEM_SHARED((NPAD,), F32),
    ],
)
def _deg_kernel(dst_hbm, deg_hbm, dst_v, ones_v, zero_v, acc):
    c = lax.axis_index("c")
    s = lax.axis_index("s")
    _fill(zero_v, RPT, 0.0)
    _fill(ones_v, EW, 1.0)
    sl = pl.ds(s * RPT, RPT)
    pltpu.sync_copy(zero_v, acc.at[sl])
    plsc.subcore_barrier()

    share = E // 32
    base = (c * 16 + s) * share

    def win(i, carry):
        pltpu.sync_copy(dst_hbm.at[pl.ds(base + i * EW, EW)], dst_v)
        pltpu.sync_copy(ones_v, acc.at[dst_v], add=True)
        return carry

    lax.fori_loop(0, share // EW, win, 0)
    plsc.subcore_barrier()
    pltpu.sync_copy(acc.at[sl], deg_hbm.at[c, sl])


@functools.partial(
    pl.kernel,
    out_type=jax.ShapeDtypeStruct((2, 2 * NPAD), F32),
    mesh=_MESH,
    compiler_params=pltpu.CompilerParams(use_tc_tiling_on_sc=False),
    scratch_types=[
        pltpu.VMEM((EW,), jnp.int32),
        pltpu.VMEM((EW,), jnp.int32),
        pltpu.VMEM((EW,), jnp.int32),
        pltpu.VMEM((EW,), F32),
        pltpu.VMEM((2 * RPT,), F32),
        pltpu.VMEM_SHARED((2 * NPAD,), F32),
    ],
)
def _l1_kernel(src_hbm, dst_hbm, xzf_hbm, t_hbm,
               src_v, dst_v, idx_v, val_v, zero_v, acc):
    c = lax.axis_index("c")
    s = lax.axis_index("s")
    _fill(zero_v, 2 * RPT, 0.0)
    sl = pl.ds(s * 2 * RPT, 2 * RPT)
    pltpu.sync_copy(zero_v, acc.at[sl])
    plsc.subcore_barrier()

    share = E // 32
    base = (c * 16 + s) * share

    def dbl(col_ref, off):
        def g(i, carry):
            v16 = col_ref[pl.ds(i * 16, 16)]
            idx_v[pl.ds(i * 16, 16)] = v16 + off
            return carry
        lax.fori_loop(0, EW // 16, g, 0)

    def win(i, carry):
        off = base + i * EW
        pltpu.sync_copy(src_hbm.at[pl.ds(off, EW)], src_v)
        pltpu.sync_copy(dst_hbm.at[pl.ds(off, EW)], dst_v)
        for col in (0, 1):
            dbl(src_v, col * NPAD)
            pltpu.sync_copy(xzf_hbm.at[idx_v], val_v)
            dbl(dst_v, col * NPAD)
            pltpu.sync_copy(val_v, acc.at[idx_v], add=True)
        return carry

    lax.fori_loop(0, share // EW, win, 0)
    plsc.subcore_barrier()
    pltpu.sync_copy(acc.at[sl], t_hbm.at[c, sl])


@functools.partial(
    pl.kernel,
    out_type=jax.ShapeDtypeStruct((2, NPAD, 16), F32),
    mesh=_MESH,
    compiler_params=pltpu.CompilerParams(use_tc_tiling_on_sc=False),
    scratch_types=[
        pltpu.VMEM((EW2,), jnp.int32),
        pltpu.VMEM((EW2,), jnp.int32),
        pltpu.VMEM((EW2,), jnp.int32),
        pltpu.VMEM((EW2, 16), F32),
        pltpu.VMEM_SHARED((NPAD, 16), F32),
    ],
)
def _l2_kernel(src_hbm, dst_hbm, z_hbm, zz_hbm, t_hbm,
               src_v, idx_v, dst_v, rows_v, acc):
    c = lax.axis_index("c")
    s = lax.axis_index("s")
    sl = pl.ds(s * RPT, RPT)
    pltpu.sync_copy(zz_hbm, acc.at[sl])
    plsc.subcore_barrier()

    share = E // 16
    base = s * share

    def win(i, carry):
        off = base + i * EW2
        pltpu.sync_copy(src_hbm.at[pl.ds(off, EW2)], src_v)
        pltpu.sync_copy(dst_hbm.at[pl.ds(off, EW2)], dst_v)

        def g(j, carry2):
            v16 = src_v[pl.ds(j * 16, 16)]
            idx_v[pl.ds(j * 16, 16)] = v16 * 2 + c
            return carry2

        lax.fori_loop(0, EW2 // 16, g, 0)
        pltpu.sync_copy(z_hbm.at[idx_v], rows_v)
        pltpu.sync_copy(rows_v, acc.at[dst_v], add=True)
        return carry

    lax.fori_loop(0, share // EW2, win, 0)
    plsc.subcore_barrier()
    pltpu.sync_copy(acc.at[sl], t_hbm.at[c, sl])


SEG = NPAD // 32
SUB = SEG // 4
ACC_N = 2050 * 16


@functools.partial(
    pl.kernel,
    out_type=jax.ShapeDtypeStruct((32, 32768), F32),
    mesh=_MESH,
    compiler_params=pltpu.CompilerParams(use_tc_tiling_on_sc=False),
    scratch_types=[
        pltpu.VMEM((SUB,), jnp.int32),
        pltpu.VMEM((SUB,), F32),
        pltpu.VMEM((SUB * 16,), F32),
        pltpu.VMEM((SUB * 16,), F32),
        pltpu.VMEM((SUB * 32,), F32),
        pltpu.VMEM((ACC_N,), F32),
        pltpu.VMEM((32,), F32),
    ],
)
def _pool_kernel(batch_hbm, dinv_hbm, t_hbm, z_hbm, bg2_hbm, p_hbm,
                 bat_v, dinv_v, ta_v, tb_v, zv, acc, bg2_v):
    c = lax.axis_index("c")
    s = lax.axis_index("s")
    w = c * 16 + s
    _fill(acc, ACC_N, -jnp.inf)
    pltpu.sync_copy(bg2_hbm, bg2_v)
    bga = bg2_v[pl.ds(0, 16)]
    bgb = bg2_v[pl.ds(16, 16)]
    nbase = w * SEG

    def sub(k, carry):
        off = nbase + k * SUB
        osl = pl.ds(off, SUB)
        fsl = pl.ds(off * 16, SUB * 16)
        fsl_b = pl.ds((NPAD + off) * 16, SUB * 16)
        pltpu.sync_copy(batch_hbm.at[osl], bat_v)
        pltpu.sync_copy(dinv_hbm.at[osl], dinv_v)
        pltpu.sync_copy(t_hbm.at[fsl], ta_v)
        pltpu.sync_copy(t_hbm.at[fsl_b], tb_v)
        pltpu.sync_copy(z_hbm.at[pl.ds(off * 32, SUB * 32)], zv)

        def grp(g, carry2):
            b16 = bat_v[pl.ds(g * 16, 16)]
            dv16 = dinv_v[pl.ds(g * 16, 16)]
            for j in range(16):
                i = g * 16 + j
                rsl = pl.ds(i * 16, 16)
                b = b16[j]
                dv = dv16[j]
                boff = b * 32
                oa = jnp.maximum(
                    dv * (ta_v[rsl] + zv[pl.ds(i * 32, 16)]) + bga, 0.0)
                acc[pl.ds(boff, 16)] = jnp.maximum(acc[pl.ds(boff, 16)], oa)
                ob = jnp.maximum(
                    dv * (tb_v[rsl] + zv[pl.ds(i * 32 + 16, 16)]) + bgb, 0.0)
                acc[pl.ds(boff + 16, 16)] = jnp.maximum(
                    acc[pl.ds(boff + 16, 16)], ob)
            return carry2

        lax.fori_loop(0, SUB // 16, grp, 0)
        return carry

    lax.fori_loop(0, SEG // SUB, sub, 0)
    pltpu.sync_copy(acc.at[pl.ds(0, 32768)], p_hbm.at[w])


def _dot(a, b):
    return lax.dot_general(a, b, (((1,), (0,)), ((), ())),
                           preferred_element_type=F32,
                           precision=lax.Precision.HIGHEST)


def _prep_body(d_ref, xc_ref, dinv_ref, xz_ref):
    d = d_ref[...]
    deg = d[0] + d[1] + 1.0
    dv = 1.0 / jnp.sqrt(deg)
    dinv_ref[...] = dv
    xc = xc_ref[...]
    xz_ref[0] = xc[0] * dv
    xz_ref[1] = xc[1] * dv


_prep = pl.pallas_call(
    _prep_body,
    grid=(98,),
    in_specs=[pl.BlockSpec((2, 8, 128), lambda i: (0, i, 0)),
              pl.BlockSpec((2, 8, 128), lambda i: (0, i, 0))],
    out_specs=[pl.BlockSpec((8, 128), lambda i: (i, 0)),
               pl.BlockSpec((2, 8, 128), lambda i: (0, i, 0))],
    out_shape=[jax.ShapeDtypeStruct((784, 128), F32),
               jax.ShapeDtypeStruct((2, 784, 128), F32)],
)


def _mid_body(t_ref, xz_ref, dinv_ref, w1_ref, b1_ref, w2_ref, z_ref):
    t = t_ref[...]
    xz = xz_ref[...]
    dv = dinv_ref[...]
    u0f = dv * (t[0, 0] + t[1, 0] + xz[0])
    u1f = dv * (t[0, 1] + t[1, 1] + xz[1])
    for s in range(8):
        u = jnp.concatenate([u0f[s:s + 1], u1f[s:s + 1]], axis=0)
        h = jnp.maximum(_dot(w1_ref[...], u) + b1_ref[...], 0.0)
        z2 = _dot(w2_ref[...], h) * dv[s:s + 1]
        z_ref[pl.ds(s * 128, 128), :] = z2.T


_mid = pl.pallas_call(
    _mid_body,
    grid=(98,),
    in_specs=[pl.BlockSpec((2, 2, 8, 128), lambda i: (0, 0, i, 0)),
              pl.BlockSpec((2, 8, 128), lambda i: (0, i, 0)),
              pl.BlockSpec((8, 128), lambda i: (i, 0)),
              pl.BlockSpec((64, 2), lambda i: (0, 0)),
              pl.BlockSpec((64, 1), lambda i: (0, 0)),
              pl.BlockSpec((32, 64), lambda i: (0, 0))],
    out_specs=pl.BlockSpec((1024, 32), lambda i: (i, 0)),
    out_shape=jax.ShapeDtypeStruct((NPAD, 32), F32),
)


def _fc_body(p_ref, wf1_ref, bf1_ref, wf2_ref, bf2_ref, o_ref, pool_ref):
    i = pl.program_id(0)

    @pl.when(i == 0)
    def _():
        pool_ref[...] = jnp.full((1024, 32), -jnp.inf, F32)

    pool_ref[...] = jnp.maximum(pool_ref[...], p_ref[...][0])

    @pl.when(i == 31)
    def _():
        o = jnp.maximum(_dot(pool_ref[...], wf1_ref[...]) + bf1_ref[...], 0.0)
        o_ref[...] = jnp.maximum(_dot(o, wf2_ref[...]) + bf2_ref[...], 0.0)


_fc = pl.pallas_call(
    _fc_body,
    grid=(32,),
    in_specs=[pl.BlockSpec((1, 1024, 32), lambda i: (i, 0, 0)),
              pl.BlockSpec((32, 32), lambda i: (0, 0)),
              pl.BlockSpec((1, 32), lambda i: (0, 0)),
              pl.BlockSpec((32, 28), lambda i: (0, 0)),
              pl.BlockSpec((1, 28), lambda i: (0, 0))],
    out_specs=pl.BlockSpec((1024, 28), lambda i: (0, 0)),
    out_shape=jax.ShapeDtypeStruct((1024, 28), F32),
    scratch_shapes=[pltpu.VMEM((1024, 32), F32)],
)


def kernel(x, edge_index, batch, Wg1, bg1, Wg2, bg2, Wf1, bf1, Wf2, bf2):
    src_e = edge_index[0]
    dst_e = edge_index[1]
    deg = _deg_kernel(dst_e)
    xcols = jnp.concatenate(
        [x.T, jnp.zeros((2, NPAD - N), F32)], axis=1).reshape(2, 784, 128)
    dinv, xz = _prep(deg.reshape(2, 784, 128), xcols)
    t1f = _l1_kernel(src_e, dst_e, xz.reshape(2 * NPAD))
    z = _mid(t1f.reshape(2, 2, 784, 128), xz, dinv, Wg1.T,
             bg1.reshape(64, 1), Wg2.T)
    zlin = z.reshape(2 * NPAD, 16)
    t2 = _l2_kernel(src_e, dst_e, zlin, jnp.zeros((RPT, 16), F32))
    batp = jnp.concatenate([batch, jnp.full((NPAD - N,), B, jnp.int32)])
    p = _pool_kernel(batp, dinv.reshape(NPAD), t2.reshape(2 * NPAD * 16),
                     zlin.reshape(NPAD * 32), bg2)
    return _fc(p.reshape(32, 1024, 32), Wf1, bf1.reshape(1, 32),
               Wf2, bf2.reshape(1, 28))


# speedup vs baseline: 48.0596x; 1.2423x over previous
import functools

import jax
import jax.numpy as jnp
from jax import lax
from jax.experimental import pallas as pl
from jax.experimental.pallas import tpu as pltpu
from jax.experimental.pallas import tpu_sc as plsc

N = 100000
E = 1600000
B = 1024
NPAD = 100352
RPT = NPAD // 16
EW = 2000
EW2 = 800
F32 = jnp.float32

_MESH = plsc.VectorSubcoreMesh(core_axis_name="c", subcore_axis_name="s")


def _fill(ref, n, value):
    vec = jnp.full((16,), value, F32)

    def body(i, carry):
        ref[pl.ds(i * 16, 16)] = vec
        return carry

    lax.fori_loop(0, n // 16, body, 0)


@functools.partial(
    pl.kernel,
    out_type=jax.ShapeDtypeStruct((2, NPAD), F32),
    mesh=_MESH,
    compiler_params=pltpu.CompilerParams(use_tc_tiling_on_sc=False),
    scratch_types=[
        pltpu.VMEM((EW,), jnp.int32),
        pltpu.VMEM((EW,), F32),
        pltpu.VMEM((RPT,), F32),
        pltpu.VMEM_SHARED((NPAD,), F32),
    ],
)
def _deg_kernel(ei_hbm, deg_hbm, dst_v, ones_v, zero_v, acc):
    c = lax.axis_index("c")
    s = lax.axis_index("s")
    _fill(zero_v, RPT, 0.0)
    _fill(ones_v, EW, 1.0)
    sl = pl.ds(s * RPT, RPT)
    pltpu.sync_copy(zero_v, acc.at[sl])
    plsc.subcore_barrier()

    share = E // 32
    base = (c * 16 + s) * share

    def win(i, carry):
        pltpu.sync_copy(ei_hbm.at[1, pl.ds(base + i * EW, EW)], dst_v)
        pltpu.sync_copy(ones_v, acc.at[dst_v], add=True)
        return carry

    lax.fori_loop(0, share // EW, win, 0)
    plsc.subcore_barrier()
    pltpu.sync_copy(acc.at[sl], deg_hbm.at[c, sl])


@functools.partial(
    pl.kernel,
    out_type=jax.ShapeDtypeStruct((2, 2 * NPAD), F32),
    mesh=_MESH,
    compiler_params=pltpu.CompilerParams(use_tc_tiling_on_sc=False),
    scratch_types=[
        pltpu.VMEM((EW,), jnp.int32),
        pltpu.VMEM((EW,), jnp.int32),
        pltpu.VMEM((EW,), jnp.int32),
        pltpu.VMEM((EW,), F32),
        pltpu.VMEM((2 * RPT,), F32),
        pltpu.VMEM_SHARED((2 * NPAD,), F32),
    ],
)
def _l1_kernel(ei_hbm, xzf_hbm, t_hbm,
               src_v, dst_v, idx_v, val_v, zero_v, acc):
    c = lax.axis_index("c")
    s = lax.axis_index("s")
    _fill(zero_v, 2 * RPT, 0.0)
    sl = pl.ds(s * 2 * RPT, 2 * RPT)
    pltpu.sync_copy(zero_v, acc.at[sl])
    plsc.subcore_barrier()

    share = E // 32
    base = (c * 16 + s) * share

    def dbl(col_ref, off):
        def g(i, carry):
            v16 = col_ref[pl.ds(i * 16, 16)]
            idx_v[pl.ds(i * 16, 16)] = v16 + off
            return carry
        lax.fori_loop(0, EW // 16, g, 0)

    def win(i, carry):
        off = base + i * EW
        pltpu.sync_copy(ei_hbm.at[0, pl.ds(off, EW)], src_v)
        pltpu.sync_copy(ei_hbm.at[1, pl.ds(off, EW)], dst_v)
        for col in (0, 1):
            dbl(src_v, col * NPAD)
            pltpu.sync_copy(xzf_hbm.at[idx_v], val_v)
            dbl(dst_v, col * NPAD)
            pltpu.sync_copy(val_v, acc.at[idx_v], add=True)
        return carry

    lax.fori_loop(0, share // EW, win, 0)
    plsc.subcore_barrier()
    pltpu.sync_copy(acc.at[sl], t_hbm.at[c, sl])


@functools.partial(
    pl.kernel,
    out_type=jax.ShapeDtypeStruct((2, NPAD, 16), F32),
    mesh=_MESH,
    compiler_params=pltpu.CompilerParams(use_tc_tiling_on_sc=False),
    scratch_types=[
        pltpu.VMEM((2, EW2), jnp.int32),
        pltpu.VMEM((2, EW2), jnp.int32),
        pltpu.VMEM((2, EW2), jnp.int32),
        pltpu.VMEM((2, EW2, 16), F32),
        pltpu.VMEM_SHARED((NPAD, 16), F32),
        pltpu.SemaphoreType.DMA,
        pltpu.SemaphoreType.DMA,
    ],
)
def _l2_kernel(ei_hbm, z_hbm, zz_hbm, t_hbm,
               src_v, idx_v, dst_v, rows_v, acc, semA, semB):
    c = lax.axis_index("c")
    s = lax.axis_index("s")
    sl = pl.ds(s * RPT, RPT)
    pltpu.sync_copy(zz_hbm, acc.at[sl])
    plsc.subcore_barrier()

    share = E // 16
    base = s * share
    nwin = share // EW2

    def stage(w, p, sem):
        off = base + w * EW2
        pltpu.async_copy(ei_hbm.at[0, pl.ds(off, EW2)], src_v.at[p], sem)
        pltpu.async_copy(ei_hbm.at[1, pl.ds(off, EW2)], dst_v.at[p], sem)

    def wait_stage(p, sem):
        pltpu.make_async_copy(ei_hbm.at[0, pl.ds(0, EW2)],
                              src_v.at[p], sem).wait()
        pltpu.make_async_copy(ei_hbm.at[1, pl.ds(0, EW2)],
                              dst_v.at[p], sem).wait()

    def process(p, sem):
        wait_stage(p, sem)

        def g(j, carry2):
            v16 = src_v[p, pl.ds(j * 16, 16)]
            idx_v[p, pl.ds(j * 16, 16)] = v16 * 2 + c
            return carry2

        lax.fori_loop(0, EW2 // 16, g, 0)
        pltpu.sync_copy(z_hbm.at[idx_v.at[p]], rows_v.at[p])
        pltpu.sync_copy(rows_v.at[p], acc.at[dst_v.at[p]], add=True)

    stage(0, 0, semA)

    def pair(ii, carry):
        w1 = 2 * ii + 1

        @pl.when(w1 < nwin)
        def _():
            stage(w1, 1, semB)

        process(0, semA)

        @pl.when(w1 < nwin)
        def _():
            @pl.when(w1 + 1 < nwin)
            def _():
                stage(w1 + 1, 0, semA)

            process(1, semB)

        return carry

    lax.fori_loop(0, (nwin + 1) // 2, pair, 0)
    plsc.subcore_barrier()
    pltpu.sync_copy(acc.at[sl], t_hbm.at[c, sl])


SEG = NPAD // 32
SUB = SEG // 4
ACC_N = 2050 * 16


@functools.partial(
    pl.kernel,
    out_type=jax.ShapeDtypeStruct((32, 32768), F32),
    mesh=_MESH,
    compiler_params=pltpu.CompilerParams(use_tc_tiling_on_sc=False),
    scratch_types=[
        pltpu.VMEM((SUB,), jnp.int32),
        pltpu.VMEM((SUB,), F32),
        pltpu.VMEM((SUB * 16,), F32),
        pltpu.VMEM((SUB * 16,), F32),
        pltpu.VMEM((SUB * 32,), F32),
        pltpu.VMEM((ACC_N,), F32),
        pltpu.VMEM((32,), F32),
    ],
)
def _pool_kernel(batch_hbm, dinv_hbm, t_hbm, z_hbm, bg2_hbm, p_hbm,
                 bat_v, dinv_v, ta_v, tb_v, zv, acc, bg2_v):
    c = lax.axis_index("c")
    s = lax.axis_index("s")
    w = c * 16 + s
    _fill(acc, ACC_N, -jnp.inf)
    pltpu.sync_copy(bg2_hbm, bg2_v)
    bga = bg2_v[pl.ds(0, 16)]
    bgb = bg2_v[pl.ds(16, 16)]
    nbase = w * SEG

    def sub(k, carry):
        off = nbase + k * SUB
        osl = pl.ds(off, SUB)
        fsl = pl.ds(off * 16, SUB * 16)
        fsl_b = pl.ds((NPAD + off) * 16, SUB * 16)
        pltpu.sync_copy(batch_hbm.at[osl], bat_v)
        pltpu.sync_copy(dinv_hbm.at[osl], dinv_v)
        pltpu.sync_copy(t_hbm.at[fsl], ta_v)
        pltpu.sync_copy(t_hbm.at[fsl_b], tb_v)
        pltpu.sync_copy(z_hbm.at[pl.ds(off * 32, SUB * 32)], zv)

        def grp(g, carry2):
            b16 = bat_v[pl.ds(g * 16, 16)]
            dv16 = dinv_v[pl.ds(g * 16, 16)]
            for j in range(16):
                i = g * 16 + j
                rsl = pl.ds(i * 16, 16)
                b = b16[j]
                dv = dv16[j]
                boff = b * 32
                oa = jnp.maximum(
                    dv * (ta_v[rsl] + zv[pl.ds(i * 32, 16)]) + bga, 0.0)
                acc[pl.ds(boff, 16)] = jnp.maximum(acc[pl.ds(boff, 16)], oa)
                ob = jnp.maximum(
                    dv * (tb_v[rsl] + zv[pl.ds(i * 32 + 16, 16)]) + bgb, 0.0)
                acc[pl.ds(boff + 16, 16)] = jnp.maximum(
                    acc[pl.ds(boff + 16, 16)], ob)
            return carry2

        lax.fori_loop(0, SUB // 16, grp, 0)
        return carry

    lax.fori_loop(0, SEG // SUB, sub, 0)
    pltpu.sync_copy(acc.at[pl.ds(0, 32768)], p_hbm.at[w])


def _dot(a, b):
    return lax.dot_general(a, b, (((1,), (0,)), ((), ())),
                           preferred_element_type=F32,
                           precision=lax.Precision.HIGHEST)


def _prep_body(d_ref, xc_ref, dinv_ref, xz_ref):
    d = d_ref[...]
    deg = d[0] + d[1] + 1.0
    dv = 1.0 / jnp.sqrt(deg)
    dinv_ref[...] = dv
    xc = xc_ref[...]
    xz_ref[0] = xc[0] * dv
    xz_ref[1] = xc[1] * dv


_prep = pl.pallas_call(
    _prep_body,
    in_specs=[pl.BlockSpec((2, 784, 128), lambda: (0, 0, 0)),
              pl.BlockSpec((2, 784, 128), lambda: (0, 0, 0))],
    out_specs=[pl.BlockSpec((784, 128), lambda: (0, 0)),
               pl.BlockSpec((2, 784, 128), lambda: (0, 0, 0))],
    out_shape=[jax.ShapeDtypeStruct((784, 128), F32),
               jax.ShapeDtypeStruct((2, 784, 128), F32)],
)


def _mid_body(t_ref, xz_ref, dinv_ref, w1_ref, b1_ref, w2_ref, z_ref):
    t = t_ref[...]
    xz = xz_ref[...]
    dv = dinv_ref[...]
    u0f = dv * (t[0, 0] + t[1, 0] + xz[0])
    u1f = dv * (t[0, 1] + t[1, 1] + xz[1])
    for s in range(8):
        u = jnp.concatenate([u0f[s:s + 1], u1f[s:s + 1]], axis=0)
        h = jnp.maximum(_dot(w1_ref[...], u) + b1_ref[...], 0.0)
        z2 = _dot(w2_ref[...], h) * dv[s:s + 1]
        z_ref[pl.ds(s * 128, 128), :] = z2.T


_mid = pl.pallas_call(
    _mid_body,
    grid=(98,),
    in_specs=[pl.BlockSpec((2, 2, 8, 128), lambda i: (0, 0, i, 0)),
              pl.BlockSpec((2, 8, 128), lambda i: (0, i, 0)),
              pl.BlockSpec((8, 128), lambda i: (i, 0)),
              pl.BlockSpec((64, 2), lambda i: (0, 0)),
              pl.BlockSpec((64, 1), lambda i: (0, 0)),
              pl.BlockSpec((32, 64), lambda i: (0, 0))],
    out_specs=pl.BlockSpec((1024, 32), lambda i: (i, 0)),
    out_shape=jax.ShapeDtypeStruct((NPAD, 32), F32),
)


def _fc_body(p_ref, wf1_ref, bf1_ref, wf2_ref, bf2_ref, o_ref, pool_ref):
    i = pl.program_id(0)

    @pl.when(i == 0)
    def _():
        pool_ref[...] = jnp.full((1024, 32), -jnp.inf, F32)

    pool_ref[...] = jnp.maximum(pool_ref[...], p_ref[...][0])

    @pl.when(i == 31)
    def _():
        o = jnp.maximum(_dot(pool_ref[...], wf1_ref[...]) + bf1_ref[...], 0.0)
        o_ref[...] = jnp.maximum(_dot(o, wf2_ref[...]) + bf2_ref[...], 0.0)


_fc = pl.pallas_call(
    _fc_body,
    grid=(32,),
    in_specs=[pl.BlockSpec((1, 1024, 32), lambda i: (i, 0, 0)),
              pl.BlockSpec((32, 32), lambda i: (0, 0)),
              pl.BlockSpec((1, 32), lambda i: (0, 0)),
              pl.BlockSpec((32, 28), lambda i: (0, 0)),
              pl.BlockSpec((1, 28), lambda i: (0, 0))],
    out_specs=pl.BlockSpec((1024, 28), lambda i: (0, 0)),
    out_shape=jax.ShapeDtypeStruct((1024, 28), F32),
    scratch_shapes=[pltpu.VMEM((1024, 32), F32)],
)


def kernel(x, edge_index, batch, Wg1, bg1, Wg2, bg2, Wf1, bf1, Wf2, bf2):
    deg = _deg_kernel(edge_index)
    xcols = jnp.concatenate(
        [x.T, jnp.zeros((2, NPAD - N), F32)], axis=1).reshape(2, 784, 128)
    dinv, xz = _prep(deg.reshape(2, 784, 128), xcols)
    t1f = _l1_kernel(edge_index, xz.reshape(2 * NPAD))
    z = _mid(t1f.reshape(2, 2, 784, 128), xz, dinv, Wg1.T,
             bg1.reshape(64, 1), Wg2.T)
    zlin = z.reshape(2 * NPAD, 16)
    t2 = _l2_kernel(edge_index, zlin, jnp.zeros((RPT, 16), F32))
    batp = jnp.concatenate([batch, jnp.full((NPAD - N,), B, jnp.int32)])
    p = _pool_kernel(batp, dinv.reshape(NPAD), t2.reshape(2 * NPAD * 16),
                     zlin.reshape(NPAD * 32), bg2)
    return _fc(p.reshape(32, 1024, 32), Wf1, bf1.reshape(1, 32),
               Wf2, bf2.reshape(1, 28))
